# Initial kernel scaffold; baseline (speedup 1.0000x reference)
import jax
import jax.numpy as jnp
from jax.experimental import pallas as pl


def kernel(x, edge_index, adj_vals, p0_pool_W, p0_pool_b, p0_W1, p0_b1, p0_W2, p0_b2, p1_pool_W, p1_pool_b, p1_W1, p1_b1, p1_W2, p1_b2):
    raise NotImplementedError("write your pallas kernel here")


# speedup vs baseline: 3.3560x; 3.3560x over previous
import functools

import jax
import jax.numpy as jnp
from jax import lax
from jax.experimental import pallas as pl
from jax.experimental.pallas import tpu as pltpu
from jax.experimental.pallas import tpu_sc as plsc

_N = 10000
_D = 256
_DH = 128
_K = 128
_NSUB = 16
_NCORE = 2
_NPAD = 10240
_ROWS_PER_TILE = _NPAD // _NSUB
_ROW_CHUNK = 128

_F32 = jnp.float32


def _l2norm_rows(h):
    n = jnp.sqrt(jnp.sum(h * h, axis=1, keepdims=True))
    return h / jnp.maximum(n, 1e-12)


def _pool_body(x_ref, wt_ref, b_ref, h_ref, h2s_ref):
    xb = _l2norm_rows(x_ref[...])
    h = jnp.dot(xb, wt_ref[...], preferred_element_type=_F32) + b_ref[...]
    h = jnp.maximum(h, 0.0)
    h_ref[...] = h
    h2 = h * h
    h2s_ref[0] = h2[:, :_DH]
    h2s_ref[1] = h2[:, _DH:]


def _mid_body(hp_ref, a0_ref, a1_ref, w1t_ref, w2ta_ref, w2tb_ref, bias_ref,
              pwt_ref, pb_ref, hp1_ref, h2s_ref):
    hp = hp_ref[...]
    out = (jnp.dot(hp, w1t_ref[...], preferred_element_type=_F32)
           + jnp.dot(jnp.sqrt(a0_ref[...]), w2ta_ref[...],
                     preferred_element_type=_F32)
           + jnp.dot(jnp.sqrt(a1_ref[...]), w2tb_ref[...],
                     preferred_element_type=_F32)
           + bias_ref[...])
    out = jnp.maximum(out, 0.0)
    h1 = _l2norm_rows(out)
    hp1 = jnp.dot(h1, pwt_ref[...], preferred_element_type=_F32) + pb_ref[...]
    hp1 = jnp.maximum(hp1, 0.0)
    hp1_ref[...] = hp1
    h2 = hp1 * hp1
    h2s_ref[0] = h2[:, :_DH]
    h2s_ref[1] = h2[:, _DH:]


def _final_body(hp_ref, a0_ref, a1_ref, w1t_ref, w2ta_ref, w2tb_ref, bias_ref,
                out_ref):
    out_ref[...] = (jnp.dot(hp_ref[...], w1t_ref[...],
                            preferred_element_type=_F32)
                    + jnp.dot(jnp.sqrt(a0_ref[...]), w2ta_ref[...],
                              preferred_element_type=_F32)
                    + jnp.dot(jnp.sqrt(a1_ref[...]), w2tb_ref[...],
                              preferred_element_type=_F32)
                    + bias_ref[...])


_R = 400
_GRID = _N // _R

_row_spec = pl.BlockSpec((_R, _D), lambda i: (i, 0))
_half_spec = pl.BlockSpec((_R, _DH), lambda i: (i, 0))
_w_spec = pl.BlockSpec((_D, _D), lambda i: (0, 0))
_wh_spec = pl.BlockSpec((_DH, _D), lambda i: (0, 0))
_b_spec = pl.BlockSpec((1, _D), lambda i: (0, 0))
_h2s_spec = pl.BlockSpec((2, _R, _DH), lambda i: (0, i, 0))


def _pool_call(x, wt, b):
    return pl.pallas_call(
        _pool_body,
        grid=(_GRID,),
        in_specs=[_row_spec, _w_spec, _b_spec],
        out_specs=[_row_spec, _h2s_spec],
        out_shape=[jax.ShapeDtypeStruct((_N, _D), _F32),
                   jax.ShapeDtypeStruct((2, _N, _DH), _F32)],
    )(x, wt, b)


def _mid_call(hp, a0, a1, w1t, w2ta, w2tb, bias, pwt, pb):
    return pl.pallas_call(
        _mid_body,
        grid=(_GRID,),
        in_specs=[_row_spec, _half_spec, _half_spec, _w_spec, _wh_spec,
                  _wh_spec, _b_spec, _w_spec, _b_spec],
        out_specs=[_row_spec, _h2s_spec],
        out_shape=[jax.ShapeDtypeStruct((_N, _D), _F32),
                   jax.ShapeDtypeStruct((2, _N, _DH), _F32)],
    )(hp, a0, a1, w1t, w2ta, w2tb, bias, pwt, pb)


def _final_call(hp, a0, a1, w1t, w2ta, w2tb, bias):
    return pl.pallas_call(
        _final_body,
        grid=(_GRID,),
        in_specs=[_row_spec, _half_spec, _half_spec, _w_spec, _wh_spec,
                  _wh_spec, _b_spec],
        out_specs=_row_spec,
        out_shape=jax.ShapeDtypeStruct((_N, _D), _F32),
    )(hp, a0, a1, w1t, w2ta, w2tb, bias)


def _sc_agg_body(src_hbm, dst_hbm, adj_hbm, h2_hbm, out_hbm,
                 idx_v, dst_v, adj_v, gbuf, acc, sem):
    c = lax.axis_index("c")
    s = lax.axis_index("s")
    nchunk = idx_v.shape[0]

    pltpu.sync_copy(src_hbm.at[s], idx_v)
    pltpu.sync_copy(dst_hbm.at[s], dst_v)
    pltpu.sync_copy(adj_hbm.at[s], adj_v)

    off = c * _N

    def _addoff(r, _):
        for q in range(8):
            idx_v[r, pl.ds(q * 16, 16)] = idx_v[r, pl.ds(q * 16, 16)] + off
        return 0

    lax.fori_loop(0, nchunk, _addoff, 0)

    def _zero(r, _):
        for q in range(8):
            gbuf[r, pl.ds(q * 16, 16)] = jnp.zeros((16,), _F32)
        return 0

    lax.fori_loop(0, _K, _zero, 0)
    base = s * _ROWS_PER_TILE
    for k in range(_ROWS_PER_TILE // _ROW_CHUNK):
        pltpu.sync_copy(gbuf.at[pl.ds(0, _ROW_CHUNK)],
                        acc.at[pl.ds(base + k * _ROW_CHUNK, _ROW_CHUNK)])
    plsc.subcore_barrier()

    def _chunk(j, _):
        pltpu.async_copy(h2_hbm.at[idx_v.at[j]], gbuf, sem).wait()

        def _grp(t, _2):
            av = adj_v[j, pl.ds(t * 16, 16)]
            for l in range(16):
                a = av[l]
                i = t * 16 + l
                for q in range(8):
                    gbuf[i, pl.ds(q * 16, 16)] = gbuf[i, pl.ds(q * 16, 16)] * a
            return 0

        lax.fori_loop(0, _K // 16, _grp, 0)
        pltpu.sync_copy(gbuf, acc.at[dst_v.at[j]], add=True)
        return 0

    lax.fori_loop(0, nchunk, _chunk, 0)
    plsc.subcore_barrier()

    for k in range(_ROWS_PER_TILE // _ROW_CHUNK):
        pltpu.sync_copy(acc.at[pl.ds(base + k * _ROW_CHUNK, _ROW_CHUNK)],
                        out_hbm.at[pl.ds(c * _NPAD + base + k * _ROW_CHUNK,
                                         _ROW_CHUNK)])


def _sc_agg(src_p, dst_p, adj_p, h2flat):
    nchunk = src_p.shape[1]
    mesh = plsc.VectorSubcoreMesh(core_axis_name="c", subcore_axis_name="s",
                                  num_cores=_NCORE, num_subcores=_NSUB)
    return pl.kernel(
        _sc_agg_body,
        mesh=mesh,
        out_type=jax.ShapeDtypeStruct((_NCORE * _NPAD, _DH), _F32),
        scratch_types=[
            pltpu.VMEM((nchunk, _K), jnp.int32),
            pltpu.VMEM((nchunk, _K), jnp.int32),
            pltpu.VMEM((nchunk, _K), _F32),
            pltpu.VMEM((_K, _DH), _F32),
            pltpu.VMEM_SHARED((_NPAD, _DH), _F32),
            pltpu.SemaphoreType.DMA,
        ],
    )(src_p, dst_p, adj_p, h2flat)


def kernel(x, edge_index, adj_vals,
           p0_pool_W, p0_pool_b, p0_W1, p0_b1, p0_W2, p0_b2,
           p1_pool_W, p1_pool_b, p1_W1, p1_b1, p1_W2, p1_b2):
    e = adj_vals.shape[0]
    ept = -(-e // (_NSUB * _K)) * _K
    pad = _NSUB * ept - e
    dst = edge_index[0]
    src = edge_index[1]
    src_p = jnp.concatenate([src, jnp.zeros((pad,), jnp.int32)])
    src_p = src_p.reshape(_NSUB, ept // _K, _K)
    dst_p = jnp.concatenate([dst, jnp.zeros((pad,), jnp.int32)])
    dst_p = dst_p.reshape(_NSUB, ept // _K, _K)
    adj_p = jnp.concatenate([adj_vals, jnp.zeros((pad,), _F32)])
    adj_p = adj_p.reshape(_NSUB, ept // _K, _K)

    b0 = (p0_b1 + p0_b2)[None, :]
    b1 = (p1_b1 + p1_b2)[None, :]

    hp0, h2s0 = _pool_call(x, p0_pool_W.T, p0_pool_b[None, :])
    agg0 = _sc_agg(src_p, dst_p, adj_p, h2s0.reshape(_NCORE * _N, _DH))
    hp1, h2s1 = _mid_call(hp0, agg0[:_N], agg0[_NPAD:_NPAD + _N],
                          p0_W1.T, p0_W2.T[:_DH], p0_W2.T[_DH:], b0,
                          p1_pool_W.T, p1_pool_b[None, :])
    agg1 = _sc_agg(src_p, dst_p, adj_p, h2s1.reshape(_NCORE * _N, _DH))
    out = _final_call(hp1, agg1[:_N], agg1[_NPAD:_NPAD + _N],
                      p1_W1.T, p1_W2.T[:_DH], p1_W2.T[_DH:], b1)
    return out
